# Initial kernel scaffold; baseline (speedup 1.0000x reference)
import jax
import jax.numpy as jnp
from jax.experimental import pallas as pl


def kernel(X, emb, W1, b1, W2, b2):
    raise NotImplementedError("write your pallas kernel here")


# speedup vs baseline: 3.8623x; 3.8623x over previous
import functools

import jax
import jax.numpy as jnp
from jax import lax
from jax.experimental import pallas as pl
from jax.experimental.pallas import tpu as pltpu
from jax.experimental.pallas import tpu_sc as plsc

_VOCAB = 819
_D = 128
_VPAD = 824

_NC = 2
_NS = 16
_NW = _NC * _NS

_B = 16384 * 26
_BPW = _B // _NW
_SUB = 128
_NSUB = _BPW // _SUB


def _mlp_table_body(emb_ref, w1t_ref, b1_ref, w2t_ref, b2_ref, out_ref):
    inv_sqrt2 = 0.7071067811865476
    x = emb_ref[...]
    x = x * 0.5 * (1.0 + lax.erf(x * inv_sqrt2))
    x = jnp.dot(x, w1t_ref[...], preferred_element_type=jnp.float32) + b1_ref[...]
    x = x * 0.5 * (1.0 + lax.erf(x * inv_sqrt2))
    x = jnp.dot(x, w2t_ref[...], preferred_element_type=jnp.float32) + b2_ref[...]
    out_ref[...] = x


def _transform_table(emb, W1, b1, W2, b2):
    emb_pad = jnp.pad(emb, ((0, _VPAD - _VOCAB), (0, 0)))
    return pl.pallas_call(
        _mlp_table_body,
        out_shape=jax.ShapeDtypeStruct((_VPAD, _D), jnp.float32),
    )(emb_pad, W1.T, b1.reshape(1, _D), W2.T, b2.reshape(1, _D))


def _gather_body(table_hbm, idx_hbm, out_hbm, idx_v, rows_v, gsem):
    c = lax.axis_index("c")
    s = lax.axis_index("s")
    wid = s * _NC + c
    base = wid * _BPW
    pltpu.sync_copy(idx_hbm.at[wid], idx_v)

    def body(j, carry):
        pltpu.async_copy(table_hbm.at[idx_v.at[j]], rows_v, gsem).wait()
        pltpu.sync_copy(rows_v, out_hbm.at[pl.ds(base + j * _SUB, _SUB)])
        return carry

    lax.fori_loop(0, _NSUB, body, 0)


@functools.lru_cache(maxsize=1)
def _gather_call():
    return pl.kernel(
        _gather_body,
        out_type=jax.ShapeDtypeStruct((_B, _D), jnp.float32),
        mesh=plsc.VectorSubcoreMesh(core_axis_name="c", subcore_axis_name="s"),
        scratch_types=[
            pltpu.VMEM((_NSUB, _SUB), jnp.int32),
            pltpu.VMEM((_SUB, _D), jnp.float32),
            pltpu.SemaphoreType.DMA,
        ],
    )


def kernel(X, emb, W1, b1, W2, b2):
    table = _transform_table(emb, W1, b1, W2, b2)
    idx = X.astype(jnp.int32).reshape(_NW, _NSUB, _SUB)
    out = _gather_call()(table, idx)
    return out.reshape(X.shape + (_D,))


# speedup vs baseline: 4.0024x; 1.0363x over previous
import functools

import jax
import jax.numpy as jnp
from jax import lax
from jax.experimental import pallas as pl
from jax.experimental.pallas import tpu as pltpu
from jax.experimental.pallas import tpu_sc as plsc

_VOCAB = 819
_D = 128
_VPAD = 824

_NC = 2
_NS = 16
_NW = _NC * _NS

_B = 16384 * 26
_BPW = _B // _NW
_SUB = 128
_NSUB = _BPW // _SUB


def _mlp_table_body(emb_ref, w1t_ref, b1_ref, w2t_ref, b2_ref, out_ref):
    inv_sqrt2 = 0.7071067811865476
    x = emb_ref[...]
    x = x * 0.5 * (1.0 + lax.erf(x * inv_sqrt2))
    x = jnp.dot(x, w1t_ref[...], preferred_element_type=jnp.float32) + b1_ref[...]
    x = x * 0.5 * (1.0 + lax.erf(x * inv_sqrt2))
    x = jnp.dot(x, w2t_ref[...], preferred_element_type=jnp.float32) + b2_ref[...]
    out_ref[...] = x


def _transform_table(emb, W1, b1, W2, b2):
    emb_pad = jnp.pad(emb, ((0, _VPAD - _VOCAB), (0, 0)))
    return pl.pallas_call(
        _mlp_table_body,
        out_shape=jax.ShapeDtypeStruct((_VPAD, _D), jnp.float32),
    )(emb_pad, W1.T, b1.reshape(1, _D), W2.T, b2.reshape(1, _D))


_NBUF = 4


def _gather_body(table_hbm, idx_hbm, out_hbm, idx_v, rows_v, gsems, ssems):
    c = lax.axis_index("c")
    s = lax.axis_index("s")
    wid = s * _NC + c
    base = wid * _BPW
    pltpu.sync_copy(idx_hbm.at[wid], idx_v)

    def gather_args(j, b):
        return table_hbm.at[idx_v.at[j]], rows_v.at[b], gsems[b]

    def scatter_args(j, b):
        return rows_v.at[b], out_hbm.at[pl.ds(base + j * _SUB, _SUB)], ssems[b]

    for b in range(_NBUF):
        pltpu.async_copy(*gather_args(b, b))

    def outer(i, carry):
        jo = i * _NBUF
        for b in range(_NBUF):
            pltpu.make_async_copy(*gather_args(jo + b, b)).wait()
            pltpu.async_copy(*scatter_args(jo + b, b))
        for b in range(_NBUF):
            jn = jo + b + _NBUF

            @pl.when(jn < _NSUB)
            def _():
                pltpu.make_async_copy(*scatter_args(jo + b, b)).wait()
                pltpu.async_copy(*gather_args(jn, b))

        return carry

    lax.fori_loop(0, _NSUB // _NBUF, outer, 0)
    for b in range(_NBUF):
        pltpu.make_async_copy(*scatter_args(_NSUB - _NBUF + b, b)).wait()


@functools.lru_cache(maxsize=1)
def _gather_call():
    return pl.kernel(
        _gather_body,
        out_type=jax.ShapeDtypeStruct((_B, _D), jnp.float32),
        mesh=plsc.VectorSubcoreMesh(core_axis_name="c", subcore_axis_name="s"),
        scratch_types=[
            pltpu.VMEM((_NSUB, _SUB), jnp.int32),
            pltpu.VMEM((_NBUF, _SUB, _D), jnp.float32),
            [pltpu.SemaphoreType.DMA] * _NBUF,
            [pltpu.SemaphoreType.DMA] * _NBUF,
        ],
    )


def kernel(X, emb, W1, b1, W2, b2):
    table = _transform_table(emb, W1, b1, W2, b2)
    idx = X.astype(jnp.int32).reshape(_NW, _NSUB, _SUB)
    out = _gather_call()(table, idx)
    return out.reshape(X.shape + (_D,))


# speedup vs baseline: 6.2277x; 1.5560x over previous
import functools

import jax
import jax.numpy as jnp
from jax import lax
from jax.experimental import pallas as pl
from jax.experimental.pallas import tpu as pltpu
from jax.experimental.pallas import tpu_sc as plsc

_VOCAB = 819
_D = 128
_VPAD = 824

_NC = 2
_NS = 16
_NW = _NC * _NS

_NBATCH = 16384
_ROWS = 26
_RPAD = 32
_BAT_PW = _NBATCH // _NW


def _mlp_table_body(emb_ref, w1t_ref, b1_ref, w2t_ref, b2_ref, out_ref):
    inv_sqrt2 = 0.7071067811865476
    x = emb_ref[...]
    x = x * 0.5 * (1.0 + lax.erf(x * inv_sqrt2))
    x = jnp.dot(x, w1t_ref[...], preferred_element_type=jnp.float32) + b1_ref[...]
    x = x * 0.5 * (1.0 + lax.erf(x * inv_sqrt2))
    x = jnp.dot(x, w2t_ref[...], preferred_element_type=jnp.float32) + b2_ref[...]
    out_ref[...] = x


def _transform_table(emb, W1, b1, W2, b2):
    emb_pad = jnp.pad(emb, ((0, _VPAD - _VOCAB), (0, 0)))
    return pl.pallas_call(
        _mlp_table_body,
        out_shape=jax.ShapeDtypeStruct((_VPAD, _D), jnp.float32),
    )(emb_pad, W1.T, b1.reshape(1, _D), W2.T, b2.reshape(1, _D))


_NBUF = 8


def _gather_body(table_hbm, idx_hbm, out_hbm, idx_v, rows_v, gsems, ssems):
    c = lax.axis_index("c")
    s = lax.axis_index("s")
    wid = s * _NC + c
    base = wid * _BAT_PW
    pltpu.sync_copy(idx_hbm.at[pl.ds(wid * _BAT_PW * _RPAD, _BAT_PW * _RPAD)], idx_v)

    def gather_args(j, b):
        return (
            table_hbm.at[idx_v.at[pl.ds(j * _RPAD, _ROWS)]],
            rows_v.at[b],
            gsems[b],
        )

    def scatter_args(j, b):
        return rows_v.at[b], out_hbm.at[base + j], ssems[b]

    for b in range(_NBUF):
        pltpu.async_copy(*gather_args(b, b))

    def outer(i, carry):
        jo = i * _NBUF
        for b in range(_NBUF):
            pltpu.make_async_copy(*gather_args(jo + b, b)).wait()
            pltpu.async_copy(*scatter_args(jo + b, b))
        for b in range(_NBUF):
            jn = jo + b + _NBUF

            @pl.when(jn < _BAT_PW)
            def _():
                pltpu.make_async_copy(*scatter_args(jo + b, b)).wait()
                pltpu.async_copy(*gather_args(jn, b))

        return carry

    lax.fori_loop(0, _BAT_PW // _NBUF, outer, 0)
    for b in range(_NBUF):
        pltpu.make_async_copy(*scatter_args(_BAT_PW - _NBUF + b, b)).wait()


@functools.lru_cache(maxsize=1)
def _gather_call():
    return pl.kernel(
        _gather_body,
        out_type=jax.ShapeDtypeStruct((_NBATCH, _ROWS, _D), jnp.float32),
        mesh=plsc.VectorSubcoreMesh(core_axis_name="c", subcore_axis_name="s"),
        scratch_types=[
            pltpu.VMEM((_BAT_PW * _RPAD,), jnp.int32),
            pltpu.VMEM((_NBUF, _ROWS, _D), jnp.float32),
            [pltpu.SemaphoreType.DMA] * _NBUF,
            [pltpu.SemaphoreType.DMA] * _NBUF,
        ],
        compiler_params=pltpu.CompilerParams(use_tc_tiling_on_sc=True),
    )


def kernel(X, emb, W1, b1, W2, b2):
    table = _transform_table(emb, W1, b1, W2, b2)
    idx = jnp.pad(X.astype(jnp.int32), ((0, 0), (0, _RPAD - _ROWS))).reshape(-1)
    return _gather_call()(table, idx)
